# baseline (speedup 1.0000x reference)
import functools

import jax
import jax.numpy as jnp
from jax import lax
from jax.experimental import pallas as pl
from jax.experimental.pallas import tpu as pltpu
from jax.experimental.pallas import tpu_sc as plsc

L = 8
NT = 1024
D = 1024
N = 4
S = 2048
ROWS = N * S
NC = 2
NS = 16
NW = NC * NS
RPW = ROWS // NW
B = 4
NB = RPW // B
LANES = 16


def _fire(w_hbm, idx_v, gath_v, sem, b, buf):
    pltpu.async_copy(
        w_hbm.at[idx_v.at[b]],
        gath_v.at[buf],
        sem,
    )


def _drain(dummy_hbm, gath_v, sem, buf):
    pltpu.make_async_copy(
        dummy_hbm,
        gath_v.at[buf],
        sem,
    ).wait()


def _accum(gath_v, outb_v, buf):
    for j in range(B):
        def cbody(c, _, j=j):
            o = pl.ds(pl.multiple_of(c * LANES, LANES), LANES)
            acc = gath_v[buf, j * L, o]
            for l in range(1, L):
                acc = acc + gath_v[buf, j * L + l, o]
            outb_v[buf, j, o] = acc
            return 0
        lax.fori_loop(0, D // LANES, cbody, 0)


def _body(x_hbm, w_hbm, dummy_hbm, out_hbm, idx_v, gath_v, outb_v, sem0, sem1):
    cid = lax.axis_index("c")
    sid = lax.axis_index("s")
    wid = sid * NC + cid
    n = wid // (S // RPW)
    s0 = (wid % (S // RPW)) * RPW
    row0 = wid * RPW

    pltpu.sync_copy(
        x_hbm.at[n, pl.ds(pl.multiple_of(s0 // B, 8), NB), :],
        idx_v,
    )

    lane = lax.iota(jnp.int32, LANES)
    loff = (lane & (L - 1)) * NT
    def off_body(bb, _):
        for k in range(L * B // LANES):
            o = pl.ds(k * LANES, LANES)
            idx_v[bb, o] = idx_v[bb, o] + loff
        return 0
    lax.fori_loop(0, NB, off_body, 0)

    def _store(b, buf):
        pltpu.sync_copy(outb_v.at[buf], out_hbm.at[pl.ds(row0 + b * B, B)])

    _fire(w_hbm, idx_v, gath_v, sem0, 0, 0)

    def outer(bb, _):
        b0 = 2 * bb
        b1 = 2 * bb + 1
        _fire(w_hbm, idx_v, gath_v, sem1, b1, 1)
        _drain(dummy_hbm, gath_v, sem0, 0)
        _accum(gath_v, outb_v, 0)
        _store(b0, 0)
        _fire(w_hbm, idx_v, gath_v, sem0, jnp.minimum(b1 + 1, NB - 1), 0)
        _drain(dummy_hbm, gath_v, sem1, 1)
        _accum(gath_v, outb_v, 1)
        _store(b1, 1)
        return 0

    lax.fori_loop(0, NB // 2, outer, 0)
    _drain(dummy_hbm, gath_v, sem0, 0)


_mek = functools.partial(
    pl.kernel,
    out_type=jax.ShapeDtypeStruct((ROWS, D), jnp.float32),
    mesh=plsc.VectorSubcoreMesh(core_axis_name="c", subcore_axis_name="s"),
    scratch_types=[
        pltpu.VMEM((NB, L * B), jnp.int32),
        pltpu.VMEM((2, L * B, D), jnp.float32),
        pltpu.VMEM((2, B, D), jnp.float32),
        pltpu.SemaphoreType.DMA,
        pltpu.SemaphoreType.DMA,
    ],
)(_body)


@jax.jit
def kernel(x, weight):
    x = x.astype(jnp.int32)
    xb = jnp.transpose(x, (0, 2, 1)).reshape(N, S // B, B * L)
    w_flat = weight.reshape(L * NT, D)
    dummy = jnp.zeros((L * B, D), jnp.float32)
    out = _mek(xb, w_flat, dummy)
    return out.reshape(N, S, D)
